# baseline (speedup 1.0000x reference)
import functools

import jax
import jax.numpy as jnp
from jax import lax
from jax.experimental import pallas as pl
from jax.experimental.pallas import tpu as pltpu
from jax.experimental.pallas import tpu_sc as plsc

N = 10000
E = 320000
F_IN = 128
F_OUT = 32
P = 12
G = 3 * F_OUT
GP = 128
N_PAD = 10240
E_PAD = 327680
ROWS_PER_TILE = N_PAD // 16
BATCH = 64
EDGES_PER_WORKER = E_PAD // 32
EDGES_PER_TILE = E_PAD // 16
NBATCH_K1 = EDGES_PER_WORKER // BATCH
NBATCH_K3 = EDGES_PER_TILE // BATCH
CHUNKS_PER_CORE = P // 2


@functools.cache
def _make_deg_kernel():
    return functools.partial(
        pl.kernel,
        mesh=plsc.VectorSubcoreMesh(core_axis_name="c", subcore_axis_name="s"),
        out_type=jax.ShapeDtypeStruct((2, N_PAD), jnp.float32),
        scratch_types=[
            pltpu.VMEM((BATCH,), jnp.float32),
            pltpu.VMEM((BATCH,), jnp.int32),
            pltpu.VMEM((ROWS_PER_TILE,), jnp.float32),
            pltpu.VMEM_SHARED((N_PAD,), jnp.float32),
        ],
    )(_deg_body)


def _deg_body(dst_hbm, out_hbm, ones_v, idx_v, zeros_v, hist_sh):
    c = lax.axis_index("c")
    s = lax.axis_index("s")
    wid = s * 2 + c

    def _zinit(j, carry):
        zeros_v[pl.ds(j * 16, 16)] = jnp.zeros((16,), jnp.float32)
        return carry

    lax.fori_loop(0, ROWS_PER_TILE // 16, _zinit, 0)

    def _oinit(j, carry):
        ones_v[pl.ds(j * 16, 16)] = jnp.full((16,), 1.0, jnp.float32)
        return carry

    lax.fori_loop(0, BATCH // 16, _oinit, 0)

    pltpu.sync_copy(zeros_v, hist_sh.at[pl.ds(s * ROWS_PER_TILE, ROWS_PER_TILE)])
    plsc.subcore_barrier()

    ebase = wid * EDGES_PER_WORKER

    def _body(i, carry):
        pltpu.sync_copy(dst_hbm.at[pl.ds(ebase + i * BATCH, BATCH)], idx_v)
        pltpu.sync_copy(ones_v, hist_sh.at[idx_v], add=True)
        return carry

    lax.fori_loop(0, NBATCH_K1, _body, 0)
    plsc.subcore_barrier()
    pltpu.sync_copy(hist_sh.at[pl.ds(s * ROWS_PER_TILE, ROWS_PER_TILE)],
                    out_hbm.at[c, pl.ds(s * ROWS_PER_TILE, ROWS_PER_TILE)])


@functools.cache
def _make_prop_kernel():
    return functools.partial(
        pl.kernel,
        mesh=plsc.VectorSubcoreMesh(core_axis_name="c", subcore_axis_name="s"),
        out_type=jax.ShapeDtypeStruct((P * N_PAD, GP), jnp.float32),
        scratch_types=[
            pltpu.VMEM((SUP, BATCH), jnp.int32),
            pltpu.VMEM((SUP, BATCH), jnp.int32),
            pltpu.VMEM((BATCH, GP), jnp.float32),
            pltpu.VMEM((BATCH, GP), jnp.float32),
            pltpu.VMEM((BATCH, GP), jnp.float32),
            pltpu.VMEM((BATCH, GP), jnp.float32),
            pltpu.VMEM_SHARED((N_PAD, GP), jnp.float32),
            pltpu.SemaphoreType.DMA,
            pltpu.SemaphoreType.DMA,
            pltpu.SemaphoreType.DMA,
            pltpu.SemaphoreType.DMA,
            pltpu.SemaphoreType.DMA,
            pltpu.SemaphoreType.DMA,
            pltpu.SemaphoreType.DMA,
            pltpu.SemaphoreType.DMA,
        ],
    )(_prop_body)


NSLOT = 4
SUP = 32
NSUP = NBATCH_K3 // SUP


def _prop_body(ys_hbm, src_hbm, dst_hbm, zeros_hbm, out_hbm,
               sidx, didx, r0, r1, r2, r3, agg_sh,
               g0, g1, g2, g3, s0, s1, s2, s3):
    c = lax.axis_index("c")
    s = lax.axis_index("s")
    rows = [r0, r1, r2, r3]
    gsems = [g0, g1, g2, g3]
    ssems = [s0, s1, s2, s3]

    def _wait_gather(t):
        pltpu.make_async_copy(ys_hbm.at[sidx.at[0]], rows[t], gsems[t]).wait()

    def _wait_scatter(t):
        pltpu.make_async_copy(rows[t], agg_sh.at[didx.at[0]], ssems[t]).wait()

    for k in range(CHUNKS_PER_CORE):
        chunk = c * CHUNKS_PER_CORE + k
        row_off = chunk * N
        pltpu.sync_copy(zeros_hbm,
                        agg_sh.at[pl.ds(s * ROWS_PER_TILE, ROWS_PER_TILE)])
        plsc.subcore_barrier()

        def _sup(u, carry):
            pltpu.sync_copy(src_hbm.at[s, u], sidx)
            pltpu.sync_copy(dst_hbm.at[s, u], didx)
            for j in range(SUP):
                for i in range(BATCH // 16):
                    sidx[j, pl.ds(i * 16, 16)] = (
                        sidx[j, pl.ds(i * 16, 16)] + row_off)
            for b in range(SUP + NSLOT):
                t = b % NSLOT
                if b >= NSLOT:
                    _wait_gather(t)
                    pltpu.async_copy(rows[t], agg_sh.at[didx.at[b - NSLOT]],
                                     ssems[t], add=True)
                if b < SUP:
                    if b >= NSLOT:
                        _wait_scatter(t)
                    pltpu.async_copy(ys_hbm.at[sidx.at[b]], rows[t], gsems[t])
            for t in range(NSLOT):
                _wait_scatter(t)
            return carry

        lax.fori_loop(0, NSUP, _sup, 0)
        plsc.subcore_barrier()
        pltpu.sync_copy(
            agg_sh.at[pl.ds(s * ROWS_PER_TILE, ROWS_PER_TILE)],
            out_hbm.at[pl.ds(chunk * N_PAD + s * ROWS_PER_TILE, ROWS_PER_TILE)])


TILE2 = 1000


def _pre_body(x_ref, w_ref, dinv_ref, ys_ref):
    dv = dinv_ref[...]
    w = w_ref[...]
    for p in range(P):
        ys_ref[p] = jnp.dot(x_ref[p], w, preferred_element_type=jnp.float32) * dv


def _pre_call(x_t, w_cat, dinv_col):
    return pl.pallas_call(
        _pre_body,
        grid=(N // TILE2,),
        in_specs=[
            pl.BlockSpec((P, TILE2, F_IN), lambda i: (0, i, 0)),
            pl.BlockSpec((F_IN, GP), lambda i: (0, 0)),
            pl.BlockSpec((TILE2, 1), lambda i: (i, 0)),
        ],
        out_specs=pl.BlockSpec((P, TILE2, GP), lambda i: (0, i, 0)),
        out_shape=jax.ShapeDtypeStruct((P, N, GP), jnp.float32),
    )(x_t, w_cat, dinv_col)


TILE4 = 1000


def _post_body(agg_ref, ys_ref, dinv_ref, att_ref, utop_ref, bzrh_ref,
               czrh_ref, uzr_ref, uh_ref, wlin_ref, blin_ref, out_ref):
    dv = dinv_ref[...]
    att = att_ref[...]
    e = jnp.exp(att - jnp.max(att))
    probs = e / jnp.sum(e)
    utop = utop_ref[...]
    bzrh = bzrh_ref[...]
    czrh = czrh_ref[...]
    uzr = uzr_ref[...]
    uh = uh_ref[...]
    h = jnp.zeros((TILE4, F_OUT), jnp.float32)
    hacc = jnp.zeros((TILE4, F_OUT), jnp.float32)
    for p in range(P):
        conv = dv * (agg_ref[p] + ys_ref[p]) + bzrh
        cp = jnp.dot(conv, utop, preferred_element_type=jnp.float32) + czrh
        zr = jax.nn.sigmoid(
            cp[:, :2 * F_OUT]
            + jnp.dot(h, uzr, preferred_element_type=jnp.float32))
        z = zr[:, :F_OUT]
        r = zr[:, F_OUT:]
        ht = jnp.tanh(
            cp[:, 2 * F_OUT:]
            + jnp.dot(h * r, uh, preferred_element_type=jnp.float32))
        h = z * h + (1.0 - z) * ht
        hacc = hacc + probs[:, p:p + 1] * h
    out_ref[...] = (jnp.dot(jax.nn.relu(hacc), wlin_ref[...],
                            preferred_element_type=jnp.float32)
                    + blin_ref[...])


def _post_call(agg, ys, dinv_col, att2, utop, bzrh, czrh, uzr, uh, wlin, blin):
    return pl.pallas_call(
        _post_body,
        grid=(N // TILE4,),
        in_specs=[
            pl.BlockSpec((P, TILE4, GP), lambda i: (0, i, 0)),
            pl.BlockSpec((P, TILE4, GP), lambda i: (0, i, 0)),
            pl.BlockSpec((TILE4, 1), lambda i: (i, 0)),
            pl.BlockSpec((1, P), lambda i: (0, 0)),
            pl.BlockSpec((GP, G), lambda i: (0, 0)),
            pl.BlockSpec((1, GP), lambda i: (0, 0)),
            pl.BlockSpec((1, G), lambda i: (0, 0)),
            pl.BlockSpec((F_OUT, 2 * F_OUT), lambda i: (0, 0)),
            pl.BlockSpec((F_OUT, F_OUT), lambda i: (0, 0)),
            pl.BlockSpec((F_OUT, P), lambda i: (0, 0)),
            pl.BlockSpec((1, P), lambda i: (0, 0)),
        ],
        out_specs=pl.BlockSpec((TILE4, P), lambda i: (i, 0)),
        out_shape=jax.ShapeDtypeStruct((N, P), jnp.float32),
    )(agg, ys, dinv_col, att2, utop, bzrh, czrh, uzr, uh, wlin, blin)


def kernel(x, edge_index, attention, W_z, b_z, W_r, b_r, W_h, b_h,
           U_z, c_z, U_r, c_r, U_h, c_h, W_lin, b_lin):
    src = edge_index[0].astype(jnp.int32)
    dst = edge_index[1].astype(jnp.int32)
    pad_e = E_PAD - E
    src_p = jnp.concatenate([src, jnp.zeros((pad_e,), jnp.int32)])
    dst_p = jnp.concatenate([dst, jnp.full((pad_e,), N_PAD - 1, jnp.int32)])

    deg2 = _make_deg_kernel()(dst_p)
    dinv = lax.rsqrt(deg2[0, :N] + deg2[1, :N] + 1.0)
    dinv_col = dinv.reshape(N, 1)

    x_t = jnp.transpose(x, (2, 0, 1))
    w_cat = jnp.concatenate(
        [W_z, W_r, W_h, jnp.zeros((F_IN, GP - G), jnp.float32)], axis=1)
    ys = _pre_call(x_t, w_cat, dinv_col)

    zeros_in = jnp.zeros((ROWS_PER_TILE, GP), jnp.float32)
    src3 = src_p.reshape(16, NSUP, SUP, BATCH)
    dst3 = dst_p.reshape(16, NSUP, SUP, BATCH)
    agg_flat = _make_prop_kernel()(ys.reshape(P * N, GP), src3, dst3, zeros_in)
    agg = agg_flat.reshape(P, N_PAD, GP)

    zb = jnp.zeros((F_OUT, F_OUT), jnp.float32)
    utop = jnp.concatenate([
        jnp.concatenate([U_z[:F_OUT], zb, zb], axis=1),
        jnp.concatenate([zb, U_r[:F_OUT], zb], axis=1),
        jnp.concatenate([zb, zb, U_h[:F_OUT]], axis=1),
        jnp.zeros((GP - G, G), jnp.float32),
    ], axis=0)
    bzrh = jnp.concatenate(
        [b_z, b_r, b_h, jnp.zeros((GP - G,), jnp.float32)]).reshape(1, GP)
    czrh = jnp.concatenate([c_z, c_r, c_h]).reshape(1, G)
    uzr = jnp.concatenate([U_z[F_OUT:], U_r[F_OUT:]], axis=1)
    uh = U_h[F_OUT:]
    att2 = attention.reshape(1, P)

    return _post_call(agg, ys, dinv_col, att2, utop, bzrh, czrh, uzr, uh,
                      W_lin, b_lin.reshape(1, P))


# speedup vs baseline: 1.0463x; 1.0463x over previous
import functools

import jax
import jax.numpy as jnp
from jax import lax
from jax.experimental import pallas as pl
from jax.experimental.pallas import tpu as pltpu
from jax.experimental.pallas import tpu_sc as plsc

N = 10000
E = 320000
F_IN = 128
F_OUT = 32
P = 12
G = 3 * F_OUT
GP = 128
N_PAD = 10240
E_PAD = 327680
ROWS_PER_TILE = N_PAD // 16
BATCH = 128
EDGES_PER_WORKER = E_PAD // 32
EDGES_PER_TILE = E_PAD // 16
NBATCH_K1 = EDGES_PER_WORKER // BATCH
NBATCH_K3 = EDGES_PER_TILE // BATCH
CHUNKS_PER_CORE = P // 2


@functools.cache
def _make_deg_kernel():
    return functools.partial(
        pl.kernel,
        mesh=plsc.VectorSubcoreMesh(core_axis_name="c", subcore_axis_name="s"),
        out_type=jax.ShapeDtypeStruct((2, N_PAD), jnp.float32),
        scratch_types=[
            pltpu.VMEM((BATCH,), jnp.float32),
            pltpu.VMEM((BATCH,), jnp.int32),
            pltpu.VMEM((ROWS_PER_TILE,), jnp.float32),
            pltpu.VMEM_SHARED((N_PAD,), jnp.float32),
        ],
    )(_deg_body)


def _deg_body(dst_hbm, out_hbm, ones_v, idx_v, zeros_v, hist_sh):
    c = lax.axis_index("c")
    s = lax.axis_index("s")
    wid = s * 2 + c

    def _zinit(j, carry):
        zeros_v[pl.ds(j * 16, 16)] = jnp.zeros((16,), jnp.float32)
        return carry

    lax.fori_loop(0, ROWS_PER_TILE // 16, _zinit, 0)

    def _oinit(j, carry):
        ones_v[pl.ds(j * 16, 16)] = jnp.full((16,), 1.0, jnp.float32)
        return carry

    lax.fori_loop(0, BATCH // 16, _oinit, 0)

    pltpu.sync_copy(zeros_v, hist_sh.at[pl.ds(s * ROWS_PER_TILE, ROWS_PER_TILE)])
    plsc.subcore_barrier()

    ebase = wid * EDGES_PER_WORKER

    def _body(i, carry):
        pltpu.sync_copy(dst_hbm.at[pl.ds(ebase + i * BATCH, BATCH)], idx_v)
        pltpu.sync_copy(ones_v, hist_sh.at[idx_v], add=True)
        return carry

    lax.fori_loop(0, NBATCH_K1, _body, 0)
    plsc.subcore_barrier()
    pltpu.sync_copy(hist_sh.at[pl.ds(s * ROWS_PER_TILE, ROWS_PER_TILE)],
                    out_hbm.at[c, pl.ds(s * ROWS_PER_TILE, ROWS_PER_TILE)])


@functools.cache
def _make_prop_kernel():
    return functools.partial(
        pl.kernel,
        mesh=plsc.VectorSubcoreMesh(core_axis_name="c", subcore_axis_name="s"),
        out_type=jax.ShapeDtypeStruct((P * N_PAD, GP), jnp.float32),
        scratch_types=[
            pltpu.VMEM((SUP, BATCH), jnp.int32),
            pltpu.VMEM((SUP, BATCH), jnp.int32),
            pltpu.VMEM((SUP, BATCH), jnp.int32),
            pltpu.VMEM((SUP, BATCH), jnp.int32),
            pltpu.VMEM((BATCH, GP), jnp.float32),
            pltpu.VMEM((BATCH, GP), jnp.float32),
            pltpu.VMEM_SHARED((N_PAD, GP), jnp.float32),
            pltpu.SemaphoreType.DMA,
            pltpu.SemaphoreType.DMA,
            pltpu.SemaphoreType.DMA,
            pltpu.SemaphoreType.DMA,
            pltpu.SemaphoreType.DMA,
            pltpu.SemaphoreType.DMA,
        ],
    )(_prop_body)


NSLOT = 2
SUP = 16
NSUP = NBATCH_K3 // SUP


def _prop_body(ys_hbm, src_hbm, dst_hbm, zeros_hbm, out_hbm,
               sidx0, sidx1, didx0, didx1, r0, r1, agg_sh,
               g0, g1, s0, s1, t0, t1):
    c = lax.axis_index("c")
    s = lax.axis_index("s")
    rows = [r0, r1]
    gsems = [g0, g1]
    ssems = [s0, s1]
    sets = [(sidx0, didx0, t0), (sidx1, didx1, t1)]

    def _stage(setidx, u):
        si, di, ts = sets[setidx]
        pltpu.async_copy(src_hbm.at[s, u], si, ts)
        pltpu.async_copy(dst_hbm.at[s, u], di, ts)

    def _wait_stage(setidx):
        si, di, ts = sets[setidx]
        pltpu.make_async_copy(src_hbm.at[s, 0], si, ts).wait()
        pltpu.make_async_copy(dst_hbm.at[s, 0], di, ts).wait()

    def _adjust(setidx, row_off):
        si, _, _ = sets[setidx]
        for j in range(SUP):
            for i in range(BATCH // 16):
                si[j, pl.ds(i * 16, 16)] = si[j, pl.ds(i * 16, 16)] + row_off

    def _wait_gather(t):
        pltpu.make_async_copy(ys_hbm.at[sidx0.at[0]], rows[t], gsems[t]).wait()

    def _wait_scatter(t):
        pltpu.make_async_copy(rows[t], agg_sh.at[didx0.at[0]], ssems[t]).wait()

    def _ring(setidx):
        si, di, _ = sets[setidx]
        for b in range(SUP + NSLOT):
            t = b % NSLOT
            if b >= NSLOT:
                _wait_gather(t)
                pltpu.async_copy(rows[t], agg_sh.at[di.at[b - NSLOT]],
                                 ssems[t], add=True)
            if b < SUP:
                if b >= NSLOT:
                    _wait_scatter(t)
                pltpu.async_copy(ys_hbm.at[si.at[b]], rows[t], gsems[t])
        for t in range(NSLOT):
            _wait_scatter(t)

    for k in range(CHUNKS_PER_CORE):
        chunk = c * CHUNKS_PER_CORE + k
        row_off = chunk * N
        _stage(0, 0)
        pltpu.sync_copy(zeros_hbm,
                        agg_sh.at[pl.ds(s * ROWS_PER_TILE, ROWS_PER_TILE)])
        plsc.subcore_barrier()

        def _grp(g, carry):
            u = 2 * g
            _wait_stage(0)
            _adjust(0, row_off)
            _stage(1, u + 1)
            _ring(0)
            _wait_stage(1)
            _adjust(1, row_off)
            pl.when(g < NSUP // 2 - 1)(lambda: _stage(0, u + 2))
            _ring(1)
            return carry

        lax.fori_loop(0, NSUP // 2, _grp, 0)
        plsc.subcore_barrier()
        pltpu.sync_copy(
            agg_sh.at[pl.ds(s * ROWS_PER_TILE, ROWS_PER_TILE)],
            out_hbm.at[pl.ds(chunk * N_PAD + s * ROWS_PER_TILE, ROWS_PER_TILE)])


TILE2 = 1000


def _pre_body(x_ref, w_ref, dinv_ref, ys_ref):
    dv = dinv_ref[...]
    w = w_ref[...]
    for p in range(P):
        ys_ref[p] = jnp.dot(x_ref[p], w, preferred_element_type=jnp.float32) * dv


def _pre_call(x_t, w_cat, dinv_col):
    return pl.pallas_call(
        _pre_body,
        grid=(N // TILE2,),
        in_specs=[
            pl.BlockSpec((P, TILE2, F_IN), lambda i: (0, i, 0)),
            pl.BlockSpec((F_IN, GP), lambda i: (0, 0)),
            pl.BlockSpec((TILE2, 1), lambda i: (i, 0)),
        ],
        out_specs=pl.BlockSpec((P, TILE2, GP), lambda i: (0, i, 0)),
        out_shape=jax.ShapeDtypeStruct((P, N, GP), jnp.float32),
    )(x_t, w_cat, dinv_col)


TILE4 = 1000


def _post_body(agg_ref, ys_ref, dinv_ref, att_ref, utop_ref, bzrh_ref,
               czrh_ref, uzr_ref, uh_ref, wlin_ref, blin_ref, out_ref):
    dv = dinv_ref[...]
    att = att_ref[...]
    e = jnp.exp(att - jnp.max(att))
    probs = e / jnp.sum(e)
    utop = utop_ref[...]
    bzrh = bzrh_ref[...]
    czrh = czrh_ref[...]
    uzr = uzr_ref[...]
    uh = uh_ref[...]
    h = jnp.zeros((TILE4, F_OUT), jnp.float32)
    hacc = jnp.zeros((TILE4, F_OUT), jnp.float32)
    for p in range(P):
        conv = dv * (agg_ref[p] + ys_ref[p]) + bzrh
        cp = jnp.dot(conv, utop, preferred_element_type=jnp.float32) + czrh
        zr = jax.nn.sigmoid(
            cp[:, :2 * F_OUT]
            + jnp.dot(h, uzr, preferred_element_type=jnp.float32))
        z = zr[:, :F_OUT]
        r = zr[:, F_OUT:]
        ht = jnp.tanh(
            cp[:, 2 * F_OUT:]
            + jnp.dot(h * r, uh, preferred_element_type=jnp.float32))
        h = z * h + (1.0 - z) * ht
        hacc = hacc + probs[:, p:p + 1] * h
    out_ref[...] = (jnp.dot(jax.nn.relu(hacc), wlin_ref[...],
                            preferred_element_type=jnp.float32)
                    + blin_ref[...])


def _post_call(agg, ys, dinv_col, att2, utop, bzrh, czrh, uzr, uh, wlin, blin):
    return pl.pallas_call(
        _post_body,
        grid=(N // TILE4,),
        in_specs=[
            pl.BlockSpec((P, TILE4, GP), lambda i: (0, i, 0)),
            pl.BlockSpec((P, TILE4, GP), lambda i: (0, i, 0)),
            pl.BlockSpec((TILE4, 1), lambda i: (i, 0)),
            pl.BlockSpec((1, P), lambda i: (0, 0)),
            pl.BlockSpec((GP, G), lambda i: (0, 0)),
            pl.BlockSpec((1, GP), lambda i: (0, 0)),
            pl.BlockSpec((1, G), lambda i: (0, 0)),
            pl.BlockSpec((F_OUT, 2 * F_OUT), lambda i: (0, 0)),
            pl.BlockSpec((F_OUT, F_OUT), lambda i: (0, 0)),
            pl.BlockSpec((F_OUT, P), lambda i: (0, 0)),
            pl.BlockSpec((1, P), lambda i: (0, 0)),
        ],
        out_specs=pl.BlockSpec((TILE4, P), lambda i: (i, 0)),
        out_shape=jax.ShapeDtypeStruct((N, P), jnp.float32),
    )(agg, ys, dinv_col, att2, utop, bzrh, czrh, uzr, uh, wlin, blin)


def kernel(x, edge_index, attention, W_z, b_z, W_r, b_r, W_h, b_h,
           U_z, c_z, U_r, c_r, U_h, c_h, W_lin, b_lin):
    src = edge_index[0].astype(jnp.int32)
    dst = edge_index[1].astype(jnp.int32)
    pad_e = E_PAD - E
    src_p = jnp.concatenate([src, jnp.zeros((pad_e,), jnp.int32)])
    dst_p = jnp.concatenate([dst, jnp.full((pad_e,), N_PAD - 1, jnp.int32)])

    deg2 = _make_deg_kernel()(dst_p)
    dinv = lax.rsqrt(deg2[0, :N] + deg2[1, :N] + 1.0)
    dinv_col = dinv.reshape(N, 1)

    x_t = jnp.transpose(x, (2, 0, 1))
    w_cat = jnp.concatenate(
        [W_z, W_r, W_h, jnp.zeros((F_IN, GP - G), jnp.float32)], axis=1)
    ys = _pre_call(x_t, w_cat, dinv_col)

    zeros_in = jnp.zeros((ROWS_PER_TILE, GP), jnp.float32)
    src3 = src_p.reshape(16, NSUP, SUP, BATCH)
    dst3 = dst_p.reshape(16, NSUP, SUP, BATCH)
    agg_flat = _make_prop_kernel()(ys.reshape(P * N, GP), src3, dst3, zeros_in)
    agg = agg_flat.reshape(P, N_PAD, GP)

    zb = jnp.zeros((F_OUT, F_OUT), jnp.float32)
    utop = jnp.concatenate([
        jnp.concatenate([U_z[:F_OUT], zb, zb], axis=1),
        jnp.concatenate([zb, U_r[:F_OUT], zb], axis=1),
        jnp.concatenate([zb, zb, U_h[:F_OUT]], axis=1),
        jnp.zeros((GP - G, G), jnp.float32),
    ], axis=0)
    bzrh = jnp.concatenate(
        [b_z, b_r, b_h, jnp.zeros((GP - G,), jnp.float32)]).reshape(1, GP)
    czrh = jnp.concatenate([c_z, c_r, c_h]).reshape(1, G)
    uzr = jnp.concatenate([U_z[F_OUT:], U_r[F_OUT:]], axis=1)
    uh = U_h[F_OUT:]
    att2 = attention.reshape(1, P)

    return _post_call(agg, ys, dinv_col, att2, utop, bzrh, czrh, uzr, uh,
                      W_lin, b_lin.reshape(1, P))
